# Initial kernel scaffold; baseline (speedup 1.0000x reference)
import jax
import jax.numpy as jnp
from jax.experimental import pallas as pl


def kernel(node_feats, edge_index, edge_feats, W, b):
    raise NotImplementedError("write your pallas kernel here")


# speedup vs baseline: 3.1490x; 3.1490x over previous
import functools

import jax
import jax.numpy as jnp
from jax import lax
from jax.experimental import pallas as pl
from jax.experimental.pallas import tpu as pltpu
from jax.experimental.pallas import tpu_sc as plsc

N = 10000
D = 128
E = 320000
EP = 327680
EDGES_PER_TILE = EP // 32
CHUNK = 128
NCHUNKS = EDGES_PER_TILE // CHUNK
NP = 10240
ROWS_PER_TILE = NP // 16


def _lane_gather(vec16, idx16):
    return lax.gather(
        vec16, idx16[:, None],
        lax.GatherDimensionNumbers(
            offset_dims=(), collapsed_slice_dims=(0,), start_index_map=(0,)),
        (1,), mode=lax.GatherScatterMode.PROMISE_IN_BOUNDS)


def _sc_gather_scale_scatter(node_feats, src, dst, w):
    mesh = plsc.VectorSubcoreMesh(core_axis_name="c", subcore_axis_name="s")

    @functools.partial(
        pl.kernel,
        mesh=mesh,
        out_type=jax.ShapeDtypeStruct((2, NP, D), jnp.float32),
        scratch_types=[
            pltpu.VMEM((CHUNK,), jnp.int32),
            pltpu.VMEM((CHUNK,), jnp.int32),
            pltpu.VMEM((CHUNK,), jnp.float32),
            pltpu.VMEM((CHUNK, D), jnp.float32),
            pltpu.VMEM((16, D), jnp.float32),
            pltpu.VMEM_SHARED((NP, D), jnp.float32),
            pltpu.SemaphoreType.DMA,
        ],
    )
    def body(nf_hbm, src_hbm, dst_hbm, w_hbm, out_hbm,
             src_v, dst_v, w_v, rows_v, zer_v, acc_sh, sem):
        c = lax.axis_index("c")
        s = lax.axis_index("s")
        wid = s * 2 + c

        zero16 = jnp.zeros((16,), jnp.float32)
        for i in range(16):
            for j in range(8):
                zer_v[i, pl.ds(j * 16, 16)] = zero16

        def zslab(t, carry):
            pltpu.sync_copy(zer_v, acc_sh.at[pl.ds(s * ROWS_PER_TILE + t * 16, 16)])
            return carry

        lax.fori_loop(0, ROWS_PER_TILE // 16, zslab, None)
        plsc.subcore_barrier()

        def chunk_body(k, carry):
            off = wid * EDGES_PER_TILE + k * CHUNK
            pltpu.sync_copy(src_hbm.at[pl.ds(off, CHUNK)], src_v)
            pltpu.sync_copy(dst_hbm.at[pl.ds(off, CHUNK)], dst_v)
            pltpu.sync_copy(w_hbm.at[pl.ds(off, CHUNK)], w_v)
            pltpu.async_copy(nf_hbm.at[src_v], rows_v, sem).wait()

            def grp(g, gc):
                w16 = w_v[pl.ds(g * 16, 16)]
                for r in range(16):
                    splat = _lane_gather(w16, jnp.full((16,), r, jnp.int32))
                    row = g * 16 + r
                    for cc in range(8):
                        rows_v[row, pl.ds(cc * 16, 16)] = (
                            rows_v[row, pl.ds(cc * 16, 16)] * splat)
                return gc

            lax.fori_loop(0, 8, grp, None)
            pltpu.sync_copy(rows_v, acc_sh.at[dst_v], add=True)
            return carry

        lax.fori_loop(0, NCHUNKS, chunk_body, None)
        plsc.subcore_barrier()
        pltpu.sync_copy(acc_sh.at[pl.ds(s * ROWS_PER_TILE, ROWS_PER_TILE)],
                        out_hbm.at[c, pl.ds(s * ROWS_PER_TILE, ROWS_PER_TILE)])

    return body(node_feats, src, dst, w)


def _tc_linear(node_feats, partials, w1t, w2t, b2d):
    br = 400

    def mm(nf_ref, p_ref, w1_ref, w2_ref, b_ref, o_ref):
        red = p_ref[0] + p_ref[1]
        o_ref[...] = (
            jnp.dot(nf_ref[...], w1_ref[...], preferred_element_type=jnp.float32)
            + jnp.dot(red, w2_ref[...], preferred_element_type=jnp.float32)
            + b_ref[...]
        )

    return pl.pallas_call(
        mm,
        grid=(N // br,),
        in_specs=[
            pl.BlockSpec((br, D), lambda i: (i, 0)),
            pl.BlockSpec((2, br, D), lambda i: (0, i, 0)),
            pl.BlockSpec((D, D), lambda i: (0, 0)),
            pl.BlockSpec((D, D), lambda i: (0, 0)),
            pl.BlockSpec((1, D), lambda i: (0, 0)),
        ],
        out_specs=pl.BlockSpec((br, D), lambda i: (i, 0)),
        out_shape=jax.ShapeDtypeStruct((N, D), jnp.float32),
    )(node_feats, partials, w1t, w2t, b2d)


def kernel(node_feats, edge_index, edge_feats, W, b):
    pad = EP - E
    src = jnp.concatenate([edge_index[0], jnp.zeros((pad,), jnp.int32)])
    dst = jnp.concatenate([edge_index[1], jnp.zeros((pad,), jnp.int32)])
    w = jnp.concatenate([edge_feats[:, 0], jnp.zeros((pad,), jnp.float32)])
    partials = _sc_gather_scale_scatter(node_feats, src, dst, w)
    w1t = W[:, :D].T
    w2t = W[:, D:].T
    return _tc_linear(node_feats, partials, w1t, w2t, b.reshape(1, D))


# speedup vs baseline: 4.5797x; 1.4543x over previous
import functools

import jax
import jax.numpy as jnp
from jax import lax
from jax.experimental import pallas as pl
from jax.experimental.pallas import tpu as pltpu
from jax.experimental.pallas import tpu_sc as plsc

N = 10000
D = 128
E = 320000
EP = 327680
EDGES_PER_TILE = EP // 32
CHUNK = 128
NCHUNKS = EDGES_PER_TILE // CHUNK
NBUF = 2
NP = 10240
ROWS_PER_TILE = NP // 16


def _lane_gather(vec16, idx16):
    return lax.gather(
        vec16, idx16[:, None],
        lax.GatherDimensionNumbers(
            offset_dims=(), collapsed_slice_dims=(0,), start_index_map=(0,)),
        (1,), mode=lax.GatherScatterMode.PROMISE_IN_BOUNDS)


def _sc_gather_scale_scatter(node_feats, enc, w):
    mesh = plsc.VectorSubcoreMesh(core_axis_name="c", subcore_axis_name="s")

    @functools.partial(
        pl.kernel,
        mesh=mesh,
        out_type=jax.ShapeDtypeStruct((2, NP, D), jnp.float32),
        scratch_types=[
            pltpu.VMEM((NCHUNKS, CHUNK), jnp.int32),
            pltpu.VMEM((NBUF, CHUNK), jnp.int32),
            pltpu.VMEM((NBUF, CHUNK), jnp.int32),
            pltpu.VMEM((NBUF, CHUNK), jnp.float32),
            pltpu.VMEM((NBUF, CHUNK, D), jnp.float32),
            pltpu.VMEM((16, D), jnp.float32),
            pltpu.VMEM_SHARED((NP, D), jnp.float32),
            pltpu.SemaphoreType.DMA,
            pltpu.SemaphoreType.DMA,
            pltpu.SemaphoreType.DMA,
            pltpu.SemaphoreType.DMA,
            pltpu.SemaphoreType.DMA,
        ],
    )
    def body(nf_hbm, enc_hbm, w_hbm, out_hbm,
             enc_v, sidx_v, didx_v, wring_v, rows_v, zer_v, acc_sh,
             isem, g0, g1, s0, s1):
        c = lax.axis_index("c")
        s = lax.axis_index("s")
        wid = s * 2 + c
        gsem = (g0, g1)
        ssem = (s0, s1)

        ld_e = pltpu.async_copy(enc_hbm.at[wid], enc_v, isem)

        zero16 = jnp.zeros((16,), jnp.float32)
        for i in range(16):
            for j in range(D // 16):
                zer_v[i, pl.ds(j * 16, 16)] = zero16

        def zslab(t, carry):
            pltpu.sync_copy(zer_v, acc_sh.at[pl.ds(s * ROWS_PER_TILE + t * 16, 16)])
            return carry

        lax.fori_loop(0, ROWS_PER_TILE // 16, zslab, None)
        ld_e.wait()
        plsc.subcore_barrier()

        def decode(k, b):
            for j in range(CHUNK // 16):
                e = enc_v[k, pl.ds(j * 16, 16)]
                sidx_v[b, pl.ds(j * 16, 16)] = e & 0xFFFF
                didx_v[b, pl.ds(j * 16, 16)] = e >> 16

        def issue_fetch(k, b):
            pltpu.async_copy(nf_hbm.at[sidx_v.at[b]], rows_v.at[b], gsem[b])
            pltpu.async_copy(w_hbm.at[wid, k], wring_v.at[b], gsem[b])

        def wait_fetch(b):
            pltpu.make_async_copy(
                nf_hbm.at[sidx_v.at[b]], rows_v.at[b], gsem[b]).wait()
            pltpu.make_async_copy(
                w_hbm.at[wid, 0], wring_v.at[b], gsem[b]).wait()

        def scale(b):
            def grp(g, gc):
                w16 = wring_v[b, pl.ds(g * 16, 16)]
                for r in range(16):
                    splat = _lane_gather(w16, jnp.full((16,), r, jnp.int32))
                    row = g * 16 + r
                    for cc in range(D // 16):
                        rows_v[b, row, pl.ds(cc * 16, 16)] = (
                            rows_v[b, row, pl.ds(cc * 16, 16)] * splat)
                return gc
            lax.fori_loop(0, CHUNK // 16, grp, None)

        decode(0, 0)
        issue_fetch(0, 0)

        def outer(k0, carry):
            for b in range(NBUF):
                k = NBUF * k0 + b
                bn = (b + 1) % NBUF

                @pl.when(k + 1 < NCHUNKS)
                def _issue():
                    @pl.when(k >= NBUF - 1)
                    def _drain():
                        pltpu.make_async_copy(
                            rows_v.at[bn], acc_sh.at[didx_v.at[bn]],
                            ssem[bn]).wait()
                    decode(k + 1, bn)
                    issue_fetch(k + 1, bn)

                wait_fetch(b)
                scale(b)
                pltpu.async_copy(
                    rows_v.at[b], acc_sh.at[didx_v.at[b]], ssem[b], add=True)
            return carry

        lax.fori_loop(0, NCHUNKS // NBUF, outer, None)
        for b in range(NBUF):
            pltpu.make_async_copy(
                rows_v.at[b], acc_sh.at[didx_v.at[b]], ssem[b]).wait()

        plsc.subcore_barrier()
        pltpu.sync_copy(acc_sh.at[pl.ds(s * ROWS_PER_TILE, ROWS_PER_TILE)],
                        out_hbm.at[c, pl.ds(s * ROWS_PER_TILE, ROWS_PER_TILE)])

    return body(node_feats, enc, w)


def _tc_linear(node_feats, partials, w1t, w2t, b2d):
    br = 400

    def mm(nf_ref, p_ref, w1_ref, w2_ref, b_ref, o_ref):
        red = p_ref[0] + p_ref[1]
        o_ref[...] = (
            jnp.dot(nf_ref[...], w1_ref[...], preferred_element_type=jnp.float32)
            + jnp.dot(red, w2_ref[...], preferred_element_type=jnp.float32)
            + b_ref[...]
        )

    return pl.pallas_call(
        mm,
        grid=(N // br,),
        in_specs=[
            pl.BlockSpec((br, D), lambda i: (i, 0)),
            pl.BlockSpec((2, br, D), lambda i: (0, i, 0)),
            pl.BlockSpec((D, D), lambda i: (0, 0)),
            pl.BlockSpec((D, D), lambda i: (0, 0)),
            pl.BlockSpec((1, D), lambda i: (0, 0)),
        ],
        out_specs=pl.BlockSpec((br, D), lambda i: (i, 0)),
        out_shape=jax.ShapeDtypeStruct((N, D), jnp.float32),
    )(node_feats, partials, w1t, w2t, b2d)


def kernel(node_feats, edge_index, edge_feats, W, b):
    pad = EP - E
    src = jnp.concatenate([edge_index[0], jnp.zeros((pad,), jnp.int32)])
    dst = jnp.concatenate([edge_index[1], jnp.zeros((pad,), jnp.int32)])
    w = jnp.concatenate([edge_feats[:, 0], jnp.zeros((pad,), jnp.float32)])
    enc = ((dst << 16) | src).reshape(32, NCHUNKS, CHUNK)
    w3 = w.reshape(32, NCHUNKS, CHUNK)
    partials = _sc_gather_scale_scatter(node_feats, enc, w3)
    w1t = W[:, :D].T
    w2t = W[:, D:].T
    return _tc_linear(node_feats, partials, w1t, w2t, b.reshape(1, D))


# speedup vs baseline: 10.5295x; 2.2992x over previous
import functools

import jax
import jax.numpy as jnp
from jax import lax
from jax.experimental import pallas as pl
from jax.experimental.pallas import tpu as pltpu
from jax.experimental.pallas import tpu_sc as plsc

N = 10000
D = 128
E = 320000
EDGES_PER_TILE = E // 32
CHUNK = 80
NCHUNKS = EDGES_PER_TILE // CHUNK
NBUF = 2
NIDX = 4
NP = 10240
ROWS_PER_TILE = NP // 16


def _lane_gather(vec16, idx16):
    return lax.gather(
        vec16, idx16[:, None],
        lax.GatherDimensionNumbers(
            offset_dims=(), collapsed_slice_dims=(0,), start_index_map=(0,)),
        (1,), mode=lax.GatherScatterMode.PROMISE_IN_BOUNDS)


def _sc_gather_scale_scatter(node_feats, ei4, w3):
    mesh = plsc.VectorSubcoreMesh(core_axis_name="c", subcore_axis_name="s")

    @functools.partial(
        pl.kernel,
        mesh=mesh,
        out_type=jax.ShapeDtypeStruct((2, NP, D), jnp.float32),
        scratch_types=[
            pltpu.VMEM((NIDX, CHUNK), jnp.int32),
            pltpu.VMEM((NIDX, CHUNK), jnp.int32),
            pltpu.VMEM((NIDX, CHUNK), jnp.float32),
            pltpu.VMEM((NBUF, CHUNK, D), jnp.float32),
            pltpu.VMEM((16, D), jnp.float32),
            pltpu.VMEM_SHARED((NP, D), jnp.float32),
            pltpu.SemaphoreType.DMA,
            pltpu.SemaphoreType.DMA,
            pltpu.SemaphoreType.DMA,
            pltpu.SemaphoreType.DMA,
            pltpu.SemaphoreType.DMA,
            pltpu.SemaphoreType.DMA,
            pltpu.SemaphoreType.DMA,
            pltpu.SemaphoreType.DMA,
        ],
    )
    def body(nf_hbm, ei_hbm, w_hbm, out_hbm,
             src_v, dst_v, wring_v, rows_v, zer_v, acc_sh,
             i0, i1, i2, i3, g0, g1, s0, s1):
        c = lax.axis_index("c")
        s = lax.axis_index("s")
        wid = s * 2 + c
        isem = (i0, i1, i2, i3)
        gsem = (g0, g1)
        ssem = (s0, s1)

        def fetch_idx(k, q):
            pltpu.async_copy(ei_hbm.at[0, wid, k], src_v.at[q], isem[q])
            pltpu.async_copy(ei_hbm.at[1, wid, k], dst_v.at[q], isem[q])
            pltpu.async_copy(w_hbm.at[wid, k], wring_v.at[q], isem[q])

        def wait_idx(q):
            pltpu.make_async_copy(ei_hbm.at[0, wid, 0], src_v.at[q], isem[q]).wait()
            pltpu.make_async_copy(ei_hbm.at[1, wid, 0], dst_v.at[q], isem[q]).wait()
            pltpu.make_async_copy(w_hbm.at[wid, 0], wring_v.at[q], isem[q]).wait()

        fetch_idx(0, 0)
        fetch_idx(1, 1)

        zero16 = jnp.zeros((16,), jnp.float32)
        for i in range(16):
            for j in range(D // 16):
                zer_v[i, pl.ds(j * 16, 16)] = zero16

        def zslab(t, carry):
            pltpu.sync_copy(zer_v, acc_sh.at[pl.ds(s * ROWS_PER_TILE + t * 16, 16)])
            return carry

        lax.fori_loop(0, ROWS_PER_TILE // 16, zslab, None)
        plsc.subcore_barrier()

        def scale(b, q):
            def grp(g, gc):
                w16 = wring_v[q, pl.ds(g * 16, 16)]
                for r in range(16):
                    splat = _lane_gather(w16, jnp.full((16,), r, jnp.int32))
                    row = g * 16 + r
                    for cc in range(D // 16):
                        rows_v[b, row, pl.ds(cc * 16, 16)] = (
                            rows_v[b, row, pl.ds(cc * 16, 16)] * splat)
                return gc
            lax.fori_loop(0, CHUNK // 16, grp, None)

        wait_idx(0)
        pltpu.async_copy(nf_hbm.at[src_v.at[0]], rows_v.at[0], gsem[0])

        def step(k, b, q, last):
            bn = (b + 1) % NBUF
            qn = (q + 1) % NIDX
            qf = (q + 2) % NIDX
            if not last:
                @pl.when(k + 2 < NCHUNKS)
                def _prefetch():
                    fetch_idx(k + 2, qf)

                @pl.when(k + 1 < NCHUNKS)
                def _issue():
                    @pl.when(k >= 1)
                    def _drain():
                        pltpu.make_async_copy(
                            rows_v.at[bn], acc_sh.at[dst_v.at[0]],
                            ssem[bn]).wait()
                    wait_idx(qn)
                    pltpu.async_copy(
                        nf_hbm.at[src_v.at[qn]], rows_v.at[bn], gsem[bn])
            pltpu.make_async_copy(
                nf_hbm.at[src_v.at[0]], rows_v.at[b], gsem[b]).wait()
            scale(b, q)
            pltpu.async_copy(
                rows_v.at[b], acc_sh.at[dst_v.at[q]], ssem[b], add=True)

        def outer(k0, carry):
            for j in range(NIDX):
                k = NIDX * k0 + j
                step(k, j % NBUF, j, False)
            return carry

        lax.fori_loop(0, NCHUNKS // NIDX, outer, None)
        step(NCHUNKS - 1, 0, 0, True)
        for b in range(NBUF):
            pltpu.make_async_copy(
                rows_v.at[b], acc_sh.at[dst_v.at[0]], ssem[b]).wait()

        plsc.subcore_barrier()
        pltpu.sync_copy(acc_sh.at[pl.ds(s * ROWS_PER_TILE, ROWS_PER_TILE)],
                        out_hbm.at[c, pl.ds(s * ROWS_PER_TILE, ROWS_PER_TILE)])

    return body(node_feats, ei4, w3)


def _tc_linear(node_feats, partials, w1t, w2t, b2d):
    br = 400

    def mm(nf_ref, p_ref, w1_ref, w2_ref, b_ref, o_ref):
        red = p_ref[0] + p_ref[1]
        o_ref[...] = (
            jnp.dot(nf_ref[...], w1_ref[...], preferred_element_type=jnp.float32)
            + jnp.dot(red, w2_ref[...], preferred_element_type=jnp.float32)
            + b_ref[...]
        )

    return pl.pallas_call(
        mm,
        grid=(N // br,),
        in_specs=[
            pl.BlockSpec((br, D), lambda i: (i, 0)),
            pl.BlockSpec((2, br, D), lambda i: (0, i, 0)),
            pl.BlockSpec((D, D), lambda i: (0, 0)),
            pl.BlockSpec((D, D), lambda i: (0, 0)),
            pl.BlockSpec((1, D), lambda i: (0, 0)),
        ],
        out_specs=pl.BlockSpec((br, D), lambda i: (i, 0)),
        out_shape=jax.ShapeDtypeStruct((N, D), jnp.float32),
    )(node_feats, partials, w1t, w2t, b2d)


def kernel(node_feats, edge_index, edge_feats, W, b):
    ei4 = edge_index.reshape(2, 32, NCHUNKS, CHUNK)
    w3 = edge_feats.reshape(32, NCHUNKS, CHUNK)
    partials = _sc_gather_scale_scatter(node_feats, ei4, w3)
    w1t = W[:, :D].T
    w2t = W[:, D:].T
    return _tc_linear(node_feats, partials, w1t, w2t, b.reshape(1, D))


# speedup vs baseline: 10.6279x; 1.0093x over previous
import functools

import jax
import jax.numpy as jnp
from jax import lax
from jax.experimental import pallas as pl
from jax.experimental.pallas import tpu as pltpu
from jax.experimental.pallas import tpu_sc as plsc

N = 10000
D = 128
E = 320000
EDGES_PER_TILE = E // 32
CHUNK = 80
NCHUNKS = EDGES_PER_TILE // CHUNK
NBUF = 2
NIDX = 4
NP = 10240
ROWS_PER_TILE = NP // 16


def _lane_gather(vec16, idx16):
    return lax.gather(
        vec16, idx16[:, None],
        lax.GatherDimensionNumbers(
            offset_dims=(), collapsed_slice_dims=(0,), start_index_map=(0,)),
        (1,), mode=lax.GatherScatterMode.PROMISE_IN_BOUNDS)


def _sc_gather_scale_scatter(node_feats, ei4, w3):
    mesh = plsc.VectorSubcoreMesh(core_axis_name="c", subcore_axis_name="s")

    @functools.partial(
        pl.kernel,
        mesh=mesh,
        out_type=jax.ShapeDtypeStruct((2, NP, D), jnp.float32),
        scratch_types=[
            pltpu.VMEM((NIDX, CHUNK), jnp.int32),
            pltpu.VMEM((NIDX, CHUNK), jnp.int32),
            pltpu.VMEM((NIDX, CHUNK), jnp.float32),
            pltpu.VMEM((NBUF, CHUNK, D), jnp.float32),
            pltpu.VMEM((16, D), jnp.float32),
            pltpu.VMEM_SHARED((NP, D), jnp.float32),
            pltpu.SemaphoreType.DMA,
            pltpu.SemaphoreType.DMA,
            pltpu.SemaphoreType.DMA,
            pltpu.SemaphoreType.DMA,
            pltpu.SemaphoreType.DMA,
            pltpu.SemaphoreType.DMA,
            pltpu.SemaphoreType.DMA,
            pltpu.SemaphoreType.DMA,
            pltpu.SemaphoreType.DMA,
        ],
    )
    def body(nf_hbm, ei_hbm, w_hbm, out_hbm,
             src_v, dst_v, wring_v, rows_v, zer_v, acc_sh,
             i0, i1, i2, i3, g0, g1, s0, s1, zsem):
        c = lax.axis_index("c")
        s = lax.axis_index("s")
        wid = s * 2 + c
        isem = (i0, i1, i2, i3)
        gsem = (g0, g1)
        ssem = (s0, s1)

        def fetch_idx(k, q):
            pltpu.async_copy(ei_hbm.at[0, wid, k], src_v.at[q], isem[q])
            pltpu.async_copy(ei_hbm.at[1, wid, k], dst_v.at[q], isem[q])
            pltpu.async_copy(w_hbm.at[wid, k], wring_v.at[q], isem[q])

        def wait_idx(q):
            pltpu.make_async_copy(ei_hbm.at[0, wid, 0], src_v.at[q], isem[q]).wait()
            pltpu.make_async_copy(ei_hbm.at[1, wid, 0], dst_v.at[q], isem[q]).wait()
            pltpu.make_async_copy(w_hbm.at[wid, 0], wring_v.at[q], isem[q]).wait()

        fetch_idx(0, 0)
        fetch_idx(1, 1)

        zero16 = jnp.zeros((16,), jnp.float32)
        for i in range(16):
            for j in range(D // 16):
                zer_v[i, pl.ds(j * 16, 16)] = zero16

        def zslab(t, carry):
            pltpu.async_copy(
                zer_v, acc_sh.at[pl.ds(s * ROWS_PER_TILE + t * 16, 16)], zsem)
            return carry

        lax.fori_loop(0, ROWS_PER_TILE // 16, zslab, None)

        def zwait(t, carry):
            pltpu.make_async_copy(
                zer_v, acc_sh.at[pl.ds(s * ROWS_PER_TILE, 16)], zsem).wait()
            return carry

        lax.fori_loop(0, ROWS_PER_TILE // 16, zwait, None)
        plsc.subcore_barrier()

        def scale(b, q):
            def grp(g, gc):
                w16 = wring_v[q, pl.ds(g * 16, 16)]
                for r in range(16):
                    splat = _lane_gather(w16, jnp.full((16,), r, jnp.int32))
                    row = g * 16 + r
                    for cc in range(D // 16):
                        rows_v[b, row, pl.ds(cc * 16, 16)] = (
                            rows_v[b, row, pl.ds(cc * 16, 16)] * splat)
                return gc
            lax.fori_loop(0, CHUNK // 16, grp, None)

        wait_idx(0)
        pltpu.async_copy(nf_hbm.at[src_v.at[0]], rows_v.at[0], gsem[0])

        def step(k, b, q, last):
            bn = (b + 1) % NBUF
            qn = (q + 1) % NIDX
            qf = (q + 2) % NIDX
            if not last:
                @pl.when(k + 2 < NCHUNKS)
                def _prefetch():
                    fetch_idx(k + 2, qf)

                @pl.when(k + 1 < NCHUNKS)
                def _issue():
                    @pl.when(k >= 1)
                    def _drain():
                        pltpu.make_async_copy(
                            rows_v.at[bn], acc_sh.at[dst_v.at[0]],
                            ssem[bn]).wait()
                    wait_idx(qn)
                    pltpu.async_copy(
                        nf_hbm.at[src_v.at[qn]], rows_v.at[bn], gsem[bn])
            pltpu.make_async_copy(
                nf_hbm.at[src_v.at[0]], rows_v.at[b], gsem[b]).wait()
            scale(b, q)
            pltpu.async_copy(
                rows_v.at[b], acc_sh.at[dst_v.at[q]], ssem[b], add=True)

        def outer(k0, carry):
            for j in range(NIDX):
                k = NIDX * k0 + j
                step(k, j % NBUF, j, False)
            return carry

        lax.fori_loop(0, NCHUNKS // NIDX, outer, None)
        step(NCHUNKS - 1, 0, 0, True)
        for b in range(NBUF):
            pltpu.make_async_copy(
                rows_v.at[b], acc_sh.at[dst_v.at[0]], ssem[b]).wait()

        plsc.subcore_barrier()
        pltpu.sync_copy(acc_sh.at[pl.ds(s * ROWS_PER_TILE, ROWS_PER_TILE)],
                        out_hbm.at[c, pl.ds(s * ROWS_PER_TILE, ROWS_PER_TILE)])

    return body(node_feats, ei4, w3)


def _tc_linear(node_feats, partials, w1t, w2t, b2d):
    br = 400

    def mm(nf_ref, p_ref, w1_ref, w2_ref, b_ref, o_ref):
        red = p_ref[0] + p_ref[1]
        o_ref[...] = (
            jnp.dot(nf_ref[...], w1_ref[...], preferred_element_type=jnp.float32)
            + jnp.dot(red, w2_ref[...], preferred_element_type=jnp.float32)
            + b_ref[...]
        )

    return pl.pallas_call(
        mm,
        grid=(N // br,),
        in_specs=[
            pl.BlockSpec((br, D), lambda i: (i, 0)),
            pl.BlockSpec((2, br, D), lambda i: (0, i, 0)),
            pl.BlockSpec((D, D), lambda i: (0, 0)),
            pl.BlockSpec((D, D), lambda i: (0, 0)),
            pl.BlockSpec((1, D), lambda i: (0, 0)),
        ],
        out_specs=pl.BlockSpec((br, D), lambda i: (i, 0)),
        out_shape=jax.ShapeDtypeStruct((N, D), jnp.float32),
    )(node_feats, partials, w1t, w2t, b2d)


def kernel(node_feats, edge_index, edge_feats, W, b):
    ei4 = edge_index.reshape(2, 32, NCHUNKS, CHUNK)
    w3 = edge_feats.reshape(32, NCHUNKS, CHUNK)
    partials = _sc_gather_scale_scatter(node_feats, ei4, w3)
    w1t = W[:, :D].T
    w2t = W[:, D:].T
    return _tc_linear(node_feats, partials, w1t, w2t, b.reshape(1, D))


# speedup vs baseline: 10.6594x; 1.0030x over previous
import functools

import jax
import jax.numpy as jnp
from jax import lax
from jax.experimental import pallas as pl
from jax.experimental.pallas import tpu as pltpu
from jax.experimental.pallas import tpu_sc as plsc

N = 10000
D = 128
E = 320000
EDGES_PER_TILE = E // 32
CHUNK = 80
NCHUNKS = EDGES_PER_TILE // CHUNK
NBUF = 2
NIDX = 4
NP = 10240
ROWS_PER_TILE = NP // 16


def _lane_gather(vec16, idx16):
    return lax.gather(
        vec16, idx16[:, None],
        lax.GatherDimensionNumbers(
            offset_dims=(), collapsed_slice_dims=(0,), start_index_map=(0,)),
        (1,), mode=lax.GatherScatterMode.PROMISE_IN_BOUNDS)


def _sc_gather_scale_scatter(node_feats, ei4, w3):
    mesh = plsc.VectorSubcoreMesh(core_axis_name="c", subcore_axis_name="s")

    @functools.partial(
        pl.kernel,
        mesh=mesh,
        out_type=jax.ShapeDtypeStruct((2, NP, D), jnp.float32),
        scratch_types=[
            pltpu.VMEM((NIDX, CHUNK), jnp.int32),
            pltpu.VMEM((NIDX, CHUNK), jnp.int32),
            pltpu.VMEM((NIDX, CHUNK), jnp.float32),
            pltpu.VMEM((NBUF, CHUNK, D), jnp.float32),
            pltpu.VMEM((16, D), jnp.float32),
            pltpu.VMEM_SHARED((NP, D), jnp.float32),
            pltpu.SemaphoreType.DMA,
            pltpu.SemaphoreType.DMA,
            pltpu.SemaphoreType.DMA,
            pltpu.SemaphoreType.DMA,
            pltpu.SemaphoreType.DMA,
            pltpu.SemaphoreType.DMA,
            pltpu.SemaphoreType.DMA,
            pltpu.SemaphoreType.DMA,
            pltpu.SemaphoreType.DMA,
        ],
    )
    def body(nf_hbm, ei_hbm, w_hbm, out_hbm,
             src_v, dst_v, wring_v, rows_v, zer_v, acc_sh,
             i0, i1, i2, i3, g0, g1, s0, s1, zsem):
        c = lax.axis_index("c")
        s = lax.axis_index("s")
        wid = s * 2 + c
        isem = (i0, i1, i2, i3)
        gsem = (g0, g1)
        ssem = (s0, s1)

        def fetch_idx(k, q):
            pltpu.async_copy(ei_hbm.at[0, wid, k], src_v.at[q], isem[q])
            pltpu.async_copy(ei_hbm.at[1, wid, k], dst_v.at[q], isem[q])
            pltpu.async_copy(w_hbm.at[wid, k], wring_v.at[q], isem[q])

        def wait_idx(q):
            pltpu.make_async_copy(ei_hbm.at[0, wid, 0], src_v.at[q], isem[q]).wait()
            pltpu.make_async_copy(ei_hbm.at[1, wid, 0], dst_v.at[q], isem[q]).wait()
            pltpu.make_async_copy(w_hbm.at[wid, 0], wring_v.at[q], isem[q]).wait()

        fetch_idx(0, 0)
        fetch_idx(1, 1)

        zero16 = jnp.zeros((16,), jnp.float32)
        for i in range(16):
            for j in range(D // 16):
                zer_v[i, pl.ds(j * 16, 16)] = zero16

        def zslab(t, carry):
            pltpu.async_copy(
                zer_v, acc_sh.at[pl.ds(s * ROWS_PER_TILE + t * 16, 16)], zsem)
            return carry

        lax.fori_loop(0, ROWS_PER_TILE // 16, zslab, None)

        def zwait(t, carry):
            pltpu.make_async_copy(
                zer_v, acc_sh.at[pl.ds(s * ROWS_PER_TILE, 16)], zsem).wait()
            return carry

        lax.fori_loop(0, ROWS_PER_TILE // 16, zwait, None)
        plsc.subcore_barrier()

        def scale(b, q):
            def grp(g, gc):
                w16 = wring_v[q, pl.ds(g * 16, 16)]
                for r in range(16):
                    splat = _lane_gather(w16, jnp.full((16,), r, jnp.int32))
                    row = g * 16 + r
                    for cc in range(D // 16):
                        rows_v[b, row, pl.ds(cc * 16, 16)] = (
                            rows_v[b, row, pl.ds(cc * 16, 16)] * splat)
                return gc
            lax.fori_loop(0, CHUNK // 16, grp, None)

        wait_idx(0)
        pltpu.async_copy(nf_hbm.at[src_v.at[0]], rows_v.at[0], gsem[0])

        def step(k, b, q, last):
            bn = (b + 1) % NBUF
            qn = (q + 1) % NIDX
            qf = (q + 2) % NIDX
            if not last:
                @pl.when(k + 2 < NCHUNKS)
                def _prefetch():
                    fetch_idx(k + 2, qf)

                @pl.when(k + 1 < NCHUNKS)
                def _issue():
                    @pl.when(k >= 1)
                    def _drain():
                        pltpu.make_async_copy(
                            rows_v.at[bn], acc_sh.at[dst_v.at[0]],
                            ssem[bn]).wait()
                    wait_idx(qn)
                    pltpu.async_copy(
                        nf_hbm.at[src_v.at[qn]], rows_v.at[bn], gsem[bn])
            pltpu.make_async_copy(
                nf_hbm.at[src_v.at[0]], rows_v.at[b], gsem[b]).wait()
            scale(b, q)
            pltpu.async_copy(
                rows_v.at[b], acc_sh.at[dst_v.at[q]], ssem[b], add=True)

        def outer(k0, carry):
            for j in range(NIDX):
                k = NIDX * k0 + j
                step(k, j % NBUF, j, False)
            return carry

        lax.fori_loop(0, NCHUNKS // NIDX, outer, None)
        step(NCHUNKS - 1, 0, 0, True)
        for b in range(NBUF):
            pltpu.make_async_copy(
                rows_v.at[b], acc_sh.at[dst_v.at[0]], ssem[b]).wait()

        plsc.subcore_barrier()
        pltpu.sync_copy(acc_sh.at[pl.ds(s * ROWS_PER_TILE, ROWS_PER_TILE)],
                        out_hbm.at[c, pl.ds(s * ROWS_PER_TILE, ROWS_PER_TILE)])

    return body(node_feats, ei4, w3)


def _tc_base(node_feats, w1t, b2d):
    br = 400

    def mm(nf_ref, w1_ref, b_ref, o_ref):
        o_ref[...] = jnp.dot(
            nf_ref[...], w1_ref[...],
            preferred_element_type=jnp.float32) + b_ref[...]

    return pl.pallas_call(
        mm,
        grid=(N // br,),
        in_specs=[
            pl.BlockSpec((br, D), lambda i: (i, 0)),
            pl.BlockSpec((D, D), lambda i: (0, 0)),
            pl.BlockSpec((1, D), lambda i: (0, 0)),
        ],
        out_specs=pl.BlockSpec((br, D), lambda i: (i, 0)),
        out_shape=jax.ShapeDtypeStruct((N, D), jnp.float32),
    )(node_feats, w1t, b2d)


def _tc_final(base, partials, w2t):
    br = 400

    def mm(base_ref, p_ref, w2_ref, o_ref):
        red = p_ref[0] + p_ref[1]
        o_ref[...] = base_ref[...] + jnp.dot(
            red, w2_ref[...], preferred_element_type=jnp.float32)

    return pl.pallas_call(
        mm,
        grid=(N // br,),
        in_specs=[
            pl.BlockSpec((br, D), lambda i: (i, 0)),
            pl.BlockSpec((2, br, D), lambda i: (0, i, 0)),
            pl.BlockSpec((D, D), lambda i: (0, 0)),
        ],
        out_specs=pl.BlockSpec((br, D), lambda i: (i, 0)),
        out_shape=jax.ShapeDtypeStruct((N, D), jnp.float32),
    )(base, partials, w2t)


def kernel(node_feats, edge_index, edge_feats, W, b):
    ei4 = edge_index.reshape(2, 32, NCHUNKS, CHUNK)
    w3 = edge_feats.reshape(32, NCHUNKS, CHUNK)
    partials = _sc_gather_scale_scatter(node_feats, ei4, w3)
    w1t = W[:, :D].T
    w2t = W[:, D:].T
    base = _tc_base(node_feats, w1t, b.reshape(1, D))
    return _tc_final(base, partials, w2t)
